# baseline (speedup 1.0000x reference)
import jax
import jax.numpy as jnp
from jax import lax
from jax.experimental import pallas as pl
from jax.experimental.pallas import tpu as pltpu
from jax.experimental.pallas import tpu_sc as plsc

_WAY = 64
_NC = 2
_NS = 16
_NW = _NC * _NS
_CHUNK = 128
_CNT_W = 16
_NBUF = 6


def _sc_segment_body(ctx_hbm, lbl_hbm, zsum_hbm, zcnt_hbm, ones_hbm,
                     sum_out, cnt_out,
                     rows_v, idx_v, ones_v, acc_s, cnt_s,
                     ld_sems, st_sems):
    cid = lax.axis_index("c")
    sid = lax.axis_index("s")
    wid = sid * _NC + cid
    n = ctx_hbm.shape[0]
    rows_per_w = n // _NW
    nchunks = rows_per_w // _CHUNK

    def start_load(k):
        base = wid * rows_per_w + k * _CHUNK
        b = k % _NBUF
        return (
            pltpu.async_copy(lbl_hbm.at[pl.ds(base, _CHUNK)], idx_v.at[b],
                             ld_sems.at[b]),
            pltpu.async_copy(ctx_hbm.at[pl.ds(base, _CHUNK), :], rows_v.at[b],
                             ld_sems.at[b]),
        )

    loads = {}
    for k in range(min(_NBUF, nchunks)):
        loads[k] = start_load(k)

    @pl.when(sid == 0)
    def _zero():
        pltpu.sync_copy(zsum_hbm, acc_s)
        pltpu.sync_copy(zcnt_hbm, cnt_s)

    pltpu.sync_copy(ones_hbm, ones_v)
    plsc.subcore_barrier()

    scats = {}
    for k in range(nchunks):
        b = k % _NBUF
        for d in loads.pop(k):
            d.wait()
        scats[k] = (
            pltpu.async_copy(rows_v.at[b], acc_s.at[idx_v.at[b]],
                             st_sems.at[b], add=True),
            pltpu.async_copy(ones_v, cnt_s.at[idx_v.at[b]],
                             st_sems.at[b], add=True),
        )
        nxt = k + _NBUF
        if nxt < nchunks:
            for d in scats.pop(k):
                d.wait()
            loads[nxt] = start_load(nxt)
    for k in sorted(scats):
        for d in scats.pop(k):
            d.wait()

    plsc.subcore_barrier()

    @pl.when(sid == 0)
    def _writeout():
        pltpu.sync_copy(acc_s, sum_out.at[cid])
        pltpu.sync_copy(cnt_s, cnt_out.at[cid])


def _tc_dist_body(sums_ref, cnts_ref, tgt_ref, out_ref):
    sums = sums_ref[0] + sums_ref[1]
    cnt = cnts_ref[0, :, 0] + cnts_ref[1, :, 0]
    protos = sums / cnt[:, None]
    t = tgt_ref[...]
    dot = lax.dot_general(t, protos, (((1,), (1,)), ((), ())),
                          preferred_element_type=jnp.float32,
                          precision=lax.Precision.HIGHEST)
    t2 = jnp.sum(t * t, axis=1, keepdims=True)
    p2 = jnp.sum(protos * protos, axis=1)
    out_ref[...] = 2.0 * dot - t2 - p2[None, :]


@jax.jit
def kernel(context_features, context_labels, target_features):
    n, d = context_features.shape
    nt = target_features.shape[0]
    labels = context_labels.astype(jnp.int32)
    zsum = jnp.zeros((_WAY, d), jnp.float32)
    zcnt = jnp.zeros((_WAY, _CNT_W), jnp.float32)
    ones = jnp.ones((_CHUNK, _CNT_W), jnp.float32)

    mesh = plsc.VectorSubcoreMesh(core_axis_name="c", subcore_axis_name="s",
                                  num_cores=_NC, num_subcores=_NS)
    sc_fn = pl.kernel(
        _sc_segment_body,
        out_type=(jax.ShapeDtypeStruct((_NC, _WAY, d), jnp.float32),
                  jax.ShapeDtypeStruct((_NC, _WAY, _CNT_W), jnp.float32)),
        mesh=mesh,
        scratch_types=[
            pltpu.VMEM((_NBUF, _CHUNK, d), jnp.float32),
            pltpu.VMEM((_NBUF, _CHUNK), jnp.int32),
            pltpu.VMEM((_CHUNK, _CNT_W), jnp.float32),
            pltpu.VMEM_SHARED((_WAY, d), jnp.float32),
            pltpu.VMEM_SHARED((_WAY, _CNT_W), jnp.float32),
            pltpu.SemaphoreType.DMA((_NBUF,)),
            pltpu.SemaphoreType.DMA((_NBUF,)),
        ],
    )
    sums, cnts = sc_fn(context_features, labels, zsum, zcnt, ones)

    tb = 1024
    logits = pl.pallas_call(
        _tc_dist_body,
        grid=(nt // tb,),
        in_specs=[
            pl.BlockSpec((_NC, _WAY, d), lambda i: (0, 0, 0)),
            pl.BlockSpec((_NC, _WAY, _CNT_W), lambda i: (0, 0, 0)),
            pl.BlockSpec((tb, d), lambda i: (i, 0)),
        ],
        out_specs=pl.BlockSpec((tb, _WAY), lambda i: (i, 0)),
        out_shape=jax.ShapeDtypeStruct((nt, _WAY), jnp.float32),
    )(sums, cnts, target_features)
    return logits


# speedup vs baseline: 1.1207x; 1.1207x over previous
import jax
import jax.numpy as jnp
from jax import lax
from jax.experimental import pallas as pl
from jax.experimental.pallas import tpu as pltpu
from jax.experimental.pallas import tpu_sc as plsc

_WAY = 64
_NC = 2
_NS = 16
_NW = _NC * _NS
_CHUNK = 128
_CNT_W = 16
_NBUF = 6


def _sc_segment_body(ctx_hbm, lbl_hbm, sum_out, cnt_out,
                     rows_v, idx_v, ones_v, zs_v, zc_v, acc_s, cnt_s,
                     ld_sems, st_sems):
    cid = lax.axis_index("c")
    sid = lax.axis_index("s")
    wid = sid * _NC + cid
    n = ctx_hbm.shape[0]
    rows_per_w = n // _NW
    nchunks = rows_per_w // _CHUNK
    stripe = _WAY // _NS

    def start_load(k):
        base = wid * rows_per_w + k * _CHUNK
        b = k % _NBUF
        return (
            pltpu.async_copy(lbl_hbm.at[pl.ds(base, _CHUNK)], idx_v.at[b],
                             ld_sems.at[b]),
            pltpu.async_copy(ctx_hbm.at[pl.ds(base, _CHUNK), :], rows_v.at[b],
                             ld_sems.at[b]),
        )

    loads = {}
    for k in range(min(_NBUF, nchunks)):
        loads[k] = start_load(k)

    zrow = jnp.zeros((16,), jnp.float32)

    def zfill(i, _):
        for j in range(8):
            zs_v[i, pl.ds(j * 16, 16)] = zrow
        zc_v[i, :] = zrow
        return 0

    lax.fori_loop(0, stripe, zfill, 0)

    def ofill(i, _):
        ones_v[i, :] = zrow + 1.0
        return 0

    lax.fori_loop(0, _CHUNK, ofill, 0)

    pltpu.sync_copy(zs_v, acc_s.at[pl.ds(sid * stripe, stripe), :])
    pltpu.sync_copy(zc_v, cnt_s.at[pl.ds(sid * stripe, stripe), :])
    plsc.subcore_barrier()

    scats = {}
    for k in range(nchunks):
        b = k % _NBUF
        nxt = k + _NBUF - 2
        if k >= 2 and nxt < nchunks:
            for d in scats.pop(k - 2):
                d.wait()
            loads[nxt] = start_load(nxt)
        for d in loads.pop(k):
            d.wait()
        scats[k] = (
            pltpu.async_copy(rows_v.at[b], acc_s.at[idx_v.at[b]],
                             st_sems.at[b], add=True),
            pltpu.async_copy(ones_v, cnt_s.at[idx_v.at[b]],
                             st_sems.at[b], add=True),
        )
    for k in sorted(scats):
        for d in scats.pop(k):
            d.wait()

    plsc.subcore_barrier()

    @pl.when(sid == 0)
    def _writeout():
        pltpu.sync_copy(acc_s, sum_out.at[cid])
        pltpu.sync_copy(cnt_s, cnt_out.at[cid])


def _tc_dist_body(sums_ref, cnts_ref, tgt_ref, out_ref):
    sums = sums_ref[0] + sums_ref[1]
    cnt = cnts_ref[0, :, 0] + cnts_ref[1, :, 0]
    protos = sums / cnt[:, None]
    t = tgt_ref[...]
    dot = lax.dot_general(t, protos, (((1,), (1,)), ((), ())),
                          preferred_element_type=jnp.float32)
    t2 = jnp.sum(t * t, axis=1, keepdims=True)
    p2 = jnp.sum(protos * protos, axis=1)
    out_ref[...] = 2.0 * dot - t2 - p2[None, :]


@jax.jit
def kernel(context_features, context_labels, target_features):
    n, d = context_features.shape
    nt = target_features.shape[0]
    labels = context_labels.astype(jnp.int32)

    mesh = plsc.VectorSubcoreMesh(core_axis_name="c", subcore_axis_name="s",
                                  num_cores=_NC, num_subcores=_NS)
    sc_fn = pl.kernel(
        _sc_segment_body,
        out_type=(jax.ShapeDtypeStruct((_NC, _WAY, d), jnp.float32),
                  jax.ShapeDtypeStruct((_NC, _WAY, _CNT_W), jnp.float32)),
        mesh=mesh,
        scratch_types=[
            pltpu.VMEM((_NBUF, _CHUNK, d), jnp.float32),
            pltpu.VMEM((_NBUF, _CHUNK), jnp.int32),
            pltpu.VMEM((_CHUNK, _CNT_W), jnp.float32),
            pltpu.VMEM((_WAY // _NS, d), jnp.float32),
            pltpu.VMEM((_WAY // _NS, _CNT_W), jnp.float32),
            pltpu.VMEM_SHARED((_WAY, d), jnp.float32),
            pltpu.VMEM_SHARED((_WAY, _CNT_W), jnp.float32),
            pltpu.SemaphoreType.DMA((_NBUF,)),
            pltpu.SemaphoreType.DMA((_NBUF,)),
        ],
    )
    sums, cnts = sc_fn(context_features, labels)

    tb = 1024
    logits = pl.pallas_call(
        _tc_dist_body,
        grid=(nt // tb,),
        in_specs=[
            pl.BlockSpec((_NC, _WAY, d), lambda i: (0, 0, 0)),
            pl.BlockSpec((_NC, _WAY, _CNT_W), lambda i: (0, 0, 0)),
            pl.BlockSpec((tb, d), lambda i: (i, 0)),
        ],
        out_specs=pl.BlockSpec((tb, _WAY), lambda i: (i, 0)),
        out_shape=jax.ShapeDtypeStruct((nt, _WAY), jnp.float32),
    )(sums, cnts, target_features)
    return logits


# speedup vs baseline: 1.2408x; 1.1071x over previous
import jax
import jax.numpy as jnp
from jax import lax
from jax.experimental import pallas as pl
from jax.experimental.pallas import tpu as pltpu
from jax.experimental.pallas import tpu_sc as plsc

_WAY = 64
_NC = 2
_NS = 16
_NW = _NC * _NS
_CHUNK = 128
_CNT_W = 16
_NBUF = 6


def _sc_segment_body(ctx_hbm, lbl_hbm, sum_out, cnt_out,
                     rows_v, idx_v, ones_v, zs_v, zc_v, acc_s, cnt_s,
                     ld_sems, st_sems):
    cid = lax.axis_index("c")
    sid = lax.axis_index("s")
    wid = sid * _NC + cid
    n = ctx_hbm.shape[0]
    rows_per_w = n // _NW
    nchunks = rows_per_w // _CHUNK
    stripe = _WAY // _NS

    def start_load(k):
        base = wid * rows_per_w + k * _CHUNK
        b = k % _NBUF
        return (
            pltpu.async_copy(lbl_hbm.at[pl.ds(base, _CHUNK)], idx_v.at[b],
                             ld_sems.at[b]),
            pltpu.async_copy(ctx_hbm.at[pl.ds(base, _CHUNK), :], rows_v.at[b],
                             ld_sems.at[b]),
        )

    loads = {}
    for k in range(min(_NBUF, nchunks)):
        loads[k] = start_load(k)

    zrow = jnp.zeros((16,), jnp.float32)

    def zfill(i, _):
        for j in range(8):
            zs_v[i, pl.ds(j * 16, 16)] = zrow
        zc_v[i, :] = zrow
        return 0

    lax.fori_loop(0, stripe, zfill, 0)

    def ofill(i, _):
        ones_v[i, :] = zrow + 1.0
        return 0

    lax.fori_loop(0, _CHUNK, ofill, 0)

    pltpu.sync_copy(zs_v, acc_s.at[pl.ds(sid * stripe, stripe), :])
    pltpu.sync_copy(zc_v, cnt_s.at[pl.ds(sid * stripe, stripe), :])
    plsc.subcore_barrier()

    scats = {}
    for k in range(nchunks):
        b = k % _NBUF
        nxt = k + _NBUF - 2
        if k >= 2 and nxt < nchunks:
            for d in scats.pop(k - 2):
                d.wait()
            loads[nxt] = start_load(nxt)
        for d in loads.pop(k):
            d.wait()
        scats[k] = (
            pltpu.async_copy(rows_v.at[b], acc_s.at[idx_v.at[b]],
                             st_sems.at[b], add=True),
            pltpu.async_copy(ones_v, cnt_s.at[idx_v.at[b]],
                             st_sems.at[b], add=True),
        )
    for k in sorted(scats):
        for d in scats.pop(k):
            d.wait()

    plsc.subcore_barrier()

    @pl.when(sid == 0)
    def _writeout():
        pltpu.sync_copy(acc_s, sum_out.at[cid])
        pltpu.sync_copy(cnt_s, cnt_out.at[cid])


def _tc_dist_body(sums_ref, cnts_ref, tgt_ref, out_ref):
    sums = sums_ref[0] + sums_ref[1]
    cnt = cnts_ref[0, :, 0] + cnts_ref[1, :, 0]
    protos = sums / cnt[:, None]
    t = tgt_ref[...]
    dot = lax.dot_general(protos, t, (((1,), (1,)), ((), ())),
                          preferred_element_type=jnp.float32)
    t2 = jnp.sum(t * t, axis=1)
    p2 = jnp.sum(protos * protos, axis=1)
    out_ref[...] = 2.0 * dot - t2[None, :] - p2[:, None]


@jax.jit
def kernel(context_features, context_labels, target_features):
    n, d = context_features.shape
    nt = target_features.shape[0]
    labels = context_labels.astype(jnp.int32)

    mesh = plsc.VectorSubcoreMesh(core_axis_name="c", subcore_axis_name="s",
                                  num_cores=_NC, num_subcores=_NS)
    sc_fn = pl.kernel(
        _sc_segment_body,
        out_type=(jax.ShapeDtypeStruct((_NC, _WAY, d), jnp.float32),
                  jax.ShapeDtypeStruct((_NC, _WAY, _CNT_W), jnp.float32)),
        mesh=mesh,
        scratch_types=[
            pltpu.VMEM((_NBUF, _CHUNK, d), jnp.float32),
            pltpu.VMEM((_NBUF, _CHUNK), jnp.int32),
            pltpu.VMEM((_CHUNK, _CNT_W), jnp.float32),
            pltpu.VMEM((_WAY // _NS, d), jnp.float32),
            pltpu.VMEM((_WAY // _NS, _CNT_W), jnp.float32),
            pltpu.VMEM_SHARED((_WAY, d), jnp.float32),
            pltpu.VMEM_SHARED((_WAY, _CNT_W), jnp.float32),
            pltpu.SemaphoreType.DMA((_NBUF,)),
            pltpu.SemaphoreType.DMA((_NBUF,)),
        ],
    )
    sums, cnts = sc_fn(context_features, labels)

    tb = 1024
    logits_t = pl.pallas_call(
        _tc_dist_body,
        grid=(nt // tb,),
        in_specs=[
            pl.BlockSpec((_NC, _WAY, d), lambda i: (0, 0, 0)),
            pl.BlockSpec((_NC, _WAY, _CNT_W), lambda i: (0, 0, 0)),
            pl.BlockSpec((tb, d), lambda i: (i, 0)),
        ],
        out_specs=pl.BlockSpec((_WAY, tb), lambda i: (0, i)),
        out_shape=jax.ShapeDtypeStruct((_WAY, nt), jnp.float32),
    )(sums, cnts, target_features)
    return logits_t.T
